# baseline (speedup 1.0000x reference)
import functools
import jax
import jax.numpy as jnp
from jax import lax
from jax.experimental import pallas as pl
from jax.experimental.pallas import tpu as pltpu
from jax.experimental.pallas import tpu_sc as plsc

_S = 1024
_CB = 32
_TB = 8
_B = 4
_C = 3
_G = 8
_HW = 512 * 512
_HALF = _HW // 2
_CHUNK = 1024
_NCHUNK = _HALF // _CHUNK
_UNROLL = 4
_CH = _C * _CB
_TH = _G * _TB

_mesh = plsc.VectorSubcoreMesh(core_axis_name="c", subcore_axis_name="s")


def _sc_hist_body(imgs_hbm, grads_hbm, rl_hbm, rs_hbm, ch_hbm, th_hbm,
                  hist_v, rs_v, rl0_v, rl1_v, vb0_v, vb1_v, sem0, sem1):
    cid = lax.axis_index("c")
    sid = lax.axis_index("s")
    wid = cid * 16 + sid

    zeros16 = jnp.zeros((16,), jnp.float32)
    ones16 = jnp.ones((16,), jnp.float32)

    bufs = [(rl0_v, vb0_v, sem0), (rl1_v, vb1_v, sem1)]

    def zero_hist(nbins):
        def zb(i, _):
            base = i * 128
            for u in range(8):
                hist_v[pl.ds(base + u * 16, 16)] = zeros16
            return ()
        lax.fori_loop(0, nbins // 128, zb, ())

    def stream_loop(b, h, nstream, rows_fn, accum_fn):
        def copies(ci, par):
            rl_v, vb_v, sem = bufs[par]
            st = h * _HALF + ci * _CHUNK
            cps = [(rl_hbm.at[b, pl.ds(st, _CHUNK)], rl_v, sem)]
            for k in range(nstream):
                ref, row = rows_fn(k)
                cps.append((ref.at[row, pl.ds(st, _CHUNK)], vb_v.at[k], sem))
            return cps

        def issue(ci, par):
            for src, dst, sem in copies(ci, par):
                pltpu.async_copy(src, dst, sem)

        def drain(ci, par):
            for src, dst, sem in copies(ci, par):
                pltpu.make_async_copy(src, dst, sem).wait()

        def process(par):
            rl_v, vb_v, _ = bufs[par]

            def jb(j, _):
                base = j * (16 * _UNROLL)
                for u in range(_UNROLL):
                    off = base + u * 16
                    r = rl_v[pl.ds(off, 16)]
                    accum_fn(r, vb_v, off)
                return ()
            lax.fori_loop(0, _CHUNK // (16 * _UNROLL), jb, ())

        issue(0, 0)

        def body(i2, _):
            ci = i2 * 2
            issue(ci + 1, 1)
            drain(ci, 0)
            process(0)
            issue(jnp.minimum(ci + 2, _NCHUNK - 1), 0)
            drain(ci + 1, 1)
            process(1)
            return ()
        lax.fori_loop(0, _NCHUNK // 2, body, ())
        drain(_NCHUNK - 1, 0)

    @pl.when(wid < 24)
    def _():
        bc = wid // 2
        h = wid % 2
        b = bc // _C
        zero_hist(_S * _TH)

        def rows(k):
            return grads_hbm, bc * _G + k

        def accum(r, vb_v, off):
            r64 = r * _TH
            for g in range(_G):
                v = vb_v[g, pl.ds(off, 16)]
                key = r64 + ((v * float(_TB - 1)).astype(jnp.int32) + g * _TB)
                plsc.addupdate_scatter(hist_v, [key], ones16)

        stream_loop(b, h, _G, rows, accum)
        pltpu.sync_copy(hist_v.at[pl.ds(0, _S * _TH)], th_hbm.at[h, bc])

    @pl.when(wid >= 24)
    def _():
        t = wid - 24
        b = t // 2
        h = t % 2
        zero_hist(_S * _CH)

        def zr(i, _):
            rs_v[pl.ds(i * 16, 16)] = zeros16
            return ()
        lax.fori_loop(0, _S // 16, zr, ())

        def rows(k):
            return imgs_hbm, b * _C + k

        def accum(r, vb_v, off):
            plsc.addupdate_scatter(rs_v, [r], ones16)
            r96 = r * _CH
            for c in range(_C):
                v = vb_v[c, pl.ds(off, 16)]
                key = r96 + ((v * float(_CB - 1)).astype(jnp.int32) + c * _CB)
                plsc.addupdate_scatter(hist_v, [key], ones16)

        stream_loop(b, h, _C, rows, accum)
        pltpu.sync_copy(hist_v, ch_hbm.at[h, b])
        pltpu.sync_copy(rs_v, rs_hbm.at[h, b])


_sc_hist = functools.partial(
    pl.kernel,
    out_type=(
        jax.ShapeDtypeStruct((2, _B, _S), jnp.float32),
        jax.ShapeDtypeStruct((2, _B, _S * _CH), jnp.float32),
        jax.ShapeDtypeStruct((2, _B * _C, _S * _TH), jnp.float32),
    ),
    mesh=_mesh,
    scratch_types=[
        pltpu.VMEM((_S * _CH,), jnp.float32),
        pltpu.VMEM((_S,), jnp.float32),
        pltpu.VMEM((_CHUNK,), jnp.int32),
        pltpu.VMEM((_CHUNK,), jnp.int32),
        pltpu.VMEM((_G, _CHUNK), jnp.float32),
        pltpu.VMEM((_G, _CHUNK), jnp.float32),
        pltpu.SemaphoreType.DMA,
        pltpu.SemaphoreType.DMA,
    ],
    compiler_params=pltpu.CompilerParams(needs_layout_passes=False),
)(_sc_hist_body)


def _epilogue_body(rs_ref, ch_ref, th_ref, rso_ref, cho_ref, tho_ref):
    rso_ref[0] = rs_ref[0, 0] + rs_ref[1, 0]
    ch = ch_ref[0, 0] + ch_ref[1, 0]
    chs = ch.sum(axis=-1, keepdims=True)
    cho_ref[0] = ch / jnp.where(chs > 0, chs, 1.0)
    tt = jnp.concatenate(
        [th_ref[0, 0, c] + th_ref[1, 0, c] for c in range(_C)],
        axis=-1)
    ths = tt.sum(axis=-1, keepdims=True)
    tho_ref[0] = tt / jnp.where(ths > 0, ths, 1.0)


_epilogue = pl.pallas_call(
    _epilogue_body,
    grid=(_B,),
    in_specs=[
        pl.BlockSpec((2, 1, 8, 128), lambda b: (0, b, 0, 0)),
        pl.BlockSpec((2, 1, _S, _CH), lambda b: (0, b, 0, 0)),
        pl.BlockSpec((2, 1, _C, _S, _TH), lambda b: (0, b, 0, 0, 0)),
    ],
    out_specs=[
        pl.BlockSpec((1, 8, 128), lambda b: (b, 0, 0)),
        pl.BlockSpec((1, _S, _CH), lambda b: (b, 0, 0)),
        pl.BlockSpec((1, _S, _C * _TH), lambda b: (b, 0, 0)),
    ],
    out_shape=[
        jax.ShapeDtypeStruct((_B, 8, 128), jnp.float32),
        jax.ShapeDtypeStruct((_B, _S, _CH), jnp.float32),
        jax.ShapeDtypeStruct((_B, _S, _C * _TH), jnp.float32),
    ],
)


def kernel(imgs, grads, reg_lab):
    B, C = imgs.shape[0], imgs.shape[1]
    G = grads.shape[2]
    imgs2 = imgs.reshape(B * C, _HW)
    grads2 = grads.reshape(B * C * G, _HW)
    rl2 = reg_lab.astype(jnp.int32).reshape(B, _HW)

    rs2, ch2, th2 = _sc_hist(imgs2, grads2, rl2)

    rs, ch, th = _epilogue(rs2.reshape(2, B, 8, 128),
                           ch2.reshape(2, B, _S, _CH),
                           th2.reshape(2, B, C, _S, _TH))
    return jnp.concatenate(
        [rs.reshape(B, _S), ch.reshape(B, -1), th.reshape(B, -1)], axis=-1)


# speedup vs baseline: 1.1180x; 1.1180x over previous
import functools
import jax
import jax.numpy as jnp
from jax import lax
from jax.experimental import pallas as pl
from jax.experimental.pallas import tpu as pltpu
from jax.experimental.pallas import tpu_sc as plsc

_S = 1024
_CB = 32
_TB = 8
_B = 4
_C = 3
_G = 8
_GP = _G // 2
_HW = 512 * 512
_HALF = _HW // 2
_CHUNK = 8192
_NCHUNK = _HALF // _CHUNK
_UNROLL = 8

_mesh = plsc.VectorSubcoreMesh(core_axis_name="c", subcore_axis_name="s")


def _sc_hist_body(imgs_hbm, grads_hbm, rl_hbm, rs_hbm, ch_hbm, th_hbm,
                  hist_v, rl0_v, rl1_v, vb0_v, vb1_v, sem0, sem1):
    cid = lax.axis_index("c")
    sid = lax.axis_index("s")
    wid = cid * 16 + sid

    zeros16 = jnp.zeros((16,), jnp.float32)
    ones16 = jnp.ones((16,), jnp.float32)

    bufs = [(rl0_v, vb0_v, sem0), (rl1_v, vb1_v, sem1)]

    def zero_hist(nbins):
        def zb(i, _):
            base = i * 128
            for u in range(8):
                hist_v[pl.ds(base + u * 16, 16)] = zeros16
            return ()
        lax.fori_loop(0, nbins // 128, zb, ())

    def stream_loop(b, h, val_rows, accum_fn):
        def start_dma(ci):
            rl_v, vb_v, sem = bufs[ci % 2]
            st = h * _HALF + ci * _CHUNK
            cps = [pltpu.async_copy(rl_hbm.at[b, pl.ds(st, _CHUNK)], rl_v, sem)]
            for k, (ref, row) in enumerate(val_rows):
                cps.append(
                    pltpu.async_copy(ref.at[row, pl.ds(st, _CHUNK)],
                                     vb_v.at[k], sem))
            return cps

        def chunk_body(rl_v, vb_v):
            def jb(j, _):
                base = j * (16 * _UNROLL)
                for u in range(_UNROLL):
                    off = base + u * 16
                    r = rl_v[pl.ds(off, 16)]
                    accum_fn(r, vb_v, off)
                return ()
            lax.fori_loop(0, _CHUNK // (16 * _UNROLL), jb, ())

        pend = start_dma(0)
        for ci in range(_NCHUNK):
            for cp in pend:
                cp.wait()
            if ci + 1 < _NCHUNK:
                pend = start_dma(ci + 1)
            rl_v, vb_v, _ = bufs[ci % 2]
            chunk_body(rl_v, vb_v)

    def tex_body(j, _):
        t = wid * 3 + j
        h = t % 2
        bcgp = t // 2
        gp = bcgp % _GP
        bc = bcgp // _GP
        b = bc // _C
        zero_hist(_S * 2 * _TB)

        def accum(r, vb_v, off):
            r16 = r * (2 * _TB)
            for k in range(2):
                v = vb_v[k, pl.ds(off, 16)]
                key = r16 + ((v * float(_TB - 1)).astype(jnp.int32) + k * _TB)
                plsc.addupdate_scatter(hist_v, [key], ones16)

        row0 = bc * _G + gp * 2
        stream_loop(b, h, [(grads_hbm, row0), (grads_hbm, row0 + 1)], accum)
        pltpu.sync_copy(hist_v.at[pl.ds(0, _S * 2 * _TB)], th_hbm.at[h, bcgp])
        return ()

    lax.fori_loop(0, 3, tex_body, ())

    @pl.when(wid < 24)
    def _():
        h = wid % 2
        bc = wid // 2
        b = bc // _C
        zero_hist(_S * _CB)

        def accum(r, vb_v, off):
            v = vb_v[0, pl.ds(off, 16)]
            key = r * _CB + (v * float(_CB - 1)).astype(jnp.int32)
            plsc.addupdate_scatter(hist_v, [key], ones16)

        stream_loop(b, h, [(imgs_hbm, bc)], accum)
        pltpu.sync_copy(hist_v, ch_hbm.at[h, bc])

    @pl.when(wid >= 24)
    def _():
        t = wid - 24
        b = t // 2
        h = t % 2
        zero_hist(_S)

        def accum(r, vb_v, off):
            plsc.addupdate_scatter(hist_v, [r], ones16)

        stream_loop(b, h, [], accum)
        pltpu.sync_copy(hist_v.at[pl.ds(0, _S)], rs_hbm.at[h, b])


_sc_hist = functools.partial(
    pl.kernel,
    out_type=(
        jax.ShapeDtypeStruct((2, _B, _S), jnp.float32),
        jax.ShapeDtypeStruct((2, _B * _C, _S * _CB), jnp.float32),
        jax.ShapeDtypeStruct((2, _B * _C * _GP, _S * 2 * _TB), jnp.float32),
    ),
    mesh=_mesh,
    scratch_types=[
        pltpu.VMEM((_S * _CB,), jnp.float32),
        pltpu.VMEM((_CHUNK,), jnp.int32),
        pltpu.VMEM((_CHUNK,), jnp.int32),
        pltpu.VMEM((2, _CHUNK), jnp.float32),
        pltpu.VMEM((2, _CHUNK), jnp.float32),
        pltpu.SemaphoreType.DMA,
        pltpu.SemaphoreType.DMA,
    ],
    compiler_params=pltpu.CompilerParams(needs_layout_passes=False),
)(_sc_hist_body)


def _epilogue_body(rs_ref, ch_ref, th_ref, rso_ref, cho_ref, tho_ref):
    rso_ref[0] = rs_ref[0, 0] + rs_ref[1, 0]
    ch = jnp.concatenate(
        [ch_ref[0, 0, c] + ch_ref[1, 0, c] for c in range(_C)],
        axis=-1)
    chs = ch.sum(axis=-1, keepdims=True)
    cho_ref[0] = ch / jnp.where(chs > 0, chs, 1.0)
    tt = jnp.concatenate(
        [th_ref[0, 0, cg] + th_ref[1, 0, cg] for cg in range(_C * _GP)],
        axis=-1)
    ths = tt.sum(axis=-1, keepdims=True)
    tho_ref[0] = tt / jnp.where(ths > 0, ths, 1.0)


_epilogue = pl.pallas_call(
    _epilogue_body,
    grid=(_B,),
    in_specs=[
        pl.BlockSpec((2, 1, 8, 128), lambda b: (0, b, 0, 0)),
        pl.BlockSpec((2, 1, _C, _S, _CB), lambda b: (0, b, 0, 0, 0)),
        pl.BlockSpec((2, 1, _C * _GP, _S, 2 * _TB), lambda b: (0, b, 0, 0, 0)),
    ],
    out_specs=[
        pl.BlockSpec((1, 8, 128), lambda b: (b, 0, 0)),
        pl.BlockSpec((1, _S, _C * _CB), lambda b: (b, 0, 0)),
        pl.BlockSpec((1, _S, _C * _G * _TB), lambda b: (b, 0, 0)),
    ],
    out_shape=[
        jax.ShapeDtypeStruct((_B, 8, 128), jnp.float32),
        jax.ShapeDtypeStruct((_B, _S, _C * _CB), jnp.float32),
        jax.ShapeDtypeStruct((_B, _S, _C * _G * _TB), jnp.float32),
    ],
)


def kernel(imgs, grads, reg_lab):
    B, C = imgs.shape[0], imgs.shape[1]
    G = grads.shape[2]
    imgs2 = imgs.reshape(B * C, _HW)
    grads2 = grads.reshape(B * C * G, _HW)
    rl2 = reg_lab.astype(jnp.int32).reshape(B, _HW)

    rs2, ch2, th2 = _sc_hist(imgs2, grads2, rl2)

    rs, ch, th = _epilogue(rs2.reshape(2, B, 8, 128),
                           ch2.reshape(2, B, C, _S, _CB),
                           th2.reshape(2, B, C * _GP, _S, 2 * _TB))
    return jnp.concatenate(
        [rs.reshape(B, _S), ch.reshape(B, -1), th.reshape(B, -1)], axis=-1)


# speedup vs baseline: 2.3656x; 2.1159x over previous
import functools
import jax
import jax.numpy as jnp
from jax import lax
from jax.experimental import pallas as pl
from jax.experimental.pallas import tpu as pltpu
from jax.experimental.pallas import tpu_sc as plsc

_S = 1024
_CB = 32
_TB = 8
_B = 4
_C = 3
_G = 8
_GP = _G // 2
_HW = 512 * 512
_HALF = _HW // 2
_CHUNK = 8192
_NCHUNK = _HALF // _CHUNK
_UNROLL = 8

_mesh = plsc.VectorSubcoreMesh(core_axis_name="c", subcore_axis_name="s")


def _sc_hist_body(imgs_hbm, grads_hbm, rl_hbm, rs_hbm, ch_hbm, th_hbm,
                  hist_v, rl0_v, rl1_v, vb0_v, vb1_v, sem0, sem1):
    cid = lax.axis_index("c")
    sid = lax.axis_index("s")
    wid = cid * 16 + sid

    zeros16 = jnp.zeros((16,), jnp.float32)
    ones16 = jnp.ones((16,), jnp.float32)

    bufs = [(rl0_v, vb0_v, sem0), (rl1_v, vb1_v, sem1)]

    def zero_hist(nbins):
        def zb(i, _):
            base = i * 128
            for u in range(8):
                hist_v[pl.ds(base + u * 16, 16)] = zeros16
            return ()
        lax.fori_loop(0, nbins // 128, zb, ())

    def stream_loop(b, h, val_rows, accum_fn):
        def start_dma(ci):
            rl_v, vb_v, sem = bufs[ci % 2]
            st = h * _HALF + ci * _CHUNK
            cps = [pltpu.async_copy(rl_hbm.at[b, pl.ds(st, _CHUNK)], rl_v, sem)]
            for k, (ref, row) in enumerate(val_rows):
                cps.append(
                    pltpu.async_copy(ref.at[row, pl.ds(st, _CHUNK)],
                                     vb_v.at[k], sem))
            return cps

        def chunk_body(rl_v, vb_v):
            def jb(j, _):
                base = j * (16 * _UNROLL)
                keyvecs = []
                for u in range(_UNROLL):
                    off = base + u * 16
                    r = rl_v[pl.ds(off, 16)]
                    keyvecs.extend(accum_fn(r, vb_v, off))
                for keys in keyvecs:
                    plsc.addupdate_scatter(hist_v, [keys], ones16)
                return ()
            lax.fori_loop(0, _CHUNK // (16 * _UNROLL), jb, ())

        pend = start_dma(0)
        for ci in range(_NCHUNK):
            for cp in pend:
                cp.wait()
            if ci + 1 < _NCHUNK:
                pend = start_dma(ci + 1)
            rl_v, vb_v, _ = bufs[ci % 2]
            chunk_body(rl_v, vb_v)

    def tex_body(j, _):
        t = wid * 3 + j
        h = t % 2
        bcgp = t // 2
        gp = bcgp % _GP
        bc = bcgp // _GP
        b = bc // _C
        zero_hist(_S * 2 * _TB)

        def accum(r, vb_v, off):
            r16 = r * (2 * _TB)
            keys = []
            for k in range(2):
                v = vb_v[k, pl.ds(off, 16)]
                keys.append(
                    r16 + ((v * float(_TB - 1)).astype(jnp.int32) + k * _TB))
            return keys

        row0 = bc * _G + gp * 2
        stream_loop(b, h, [(grads_hbm, row0), (grads_hbm, row0 + 1)], accum)
        pltpu.sync_copy(hist_v.at[pl.ds(0, _S * 2 * _TB)], th_hbm.at[h, bcgp])
        return ()

    lax.fori_loop(0, 3, tex_body, ())

    @pl.when(wid < 24)
    def _():
        h = wid % 2
        bc = wid // 2
        b = bc // _C
        zero_hist(_S * _CB)

        def accum(r, vb_v, off):
            v = vb_v[0, pl.ds(off, 16)]
            return [r * _CB + (v * float(_CB - 1)).astype(jnp.int32)]

        stream_loop(b, h, [(imgs_hbm, bc)], accum)
        pltpu.sync_copy(hist_v, ch_hbm.at[h, bc])

    @pl.when(wid >= 24)
    def _():
        t = wid - 24
        b = t // 2
        h = t % 2
        zero_hist(_S)

        def accum(r, vb_v, off):
            return [r]

        stream_loop(b, h, [], accum)
        pltpu.sync_copy(hist_v.at[pl.ds(0, _S)], rs_hbm.at[h, b])


_sc_hist = functools.partial(
    pl.kernel,
    out_type=(
        jax.ShapeDtypeStruct((2, _B, _S), jnp.float32),
        jax.ShapeDtypeStruct((2, _B * _C, _S * _CB), jnp.float32),
        jax.ShapeDtypeStruct((2, _B * _C * _GP, _S * 2 * _TB), jnp.float32),
    ),
    mesh=_mesh,
    scratch_types=[
        pltpu.VMEM((_S * _CB,), jnp.float32),
        pltpu.VMEM((_CHUNK,), jnp.int32),
        pltpu.VMEM((_CHUNK,), jnp.int32),
        pltpu.VMEM((2, _CHUNK), jnp.float32),
        pltpu.VMEM((2, _CHUNK), jnp.float32),
        pltpu.SemaphoreType.DMA,
        pltpu.SemaphoreType.DMA,
    ],
    compiler_params=pltpu.CompilerParams(needs_layout_passes=False),
)(_sc_hist_body)


def _epilogue_body(rs_ref, ch_ref, th_ref, rso_ref, cho_ref, tho_ref):
    rso_ref[0] = rs_ref[0, 0] + rs_ref[1, 0]
    ch = jnp.concatenate(
        [ch_ref[0, 0, c] + ch_ref[1, 0, c] for c in range(_C)],
        axis=-1)
    chs = ch.sum(axis=-1, keepdims=True)
    cho_ref[0] = ch / jnp.where(chs > 0, chs, 1.0)
    tt = jnp.concatenate(
        [th_ref[0, 0, cg] + th_ref[1, 0, cg] for cg in range(_C * _GP)],
        axis=-1)
    ths = tt.sum(axis=-1, keepdims=True)
    tho_ref[0] = tt / jnp.where(ths > 0, ths, 1.0)


_epilogue = pl.pallas_call(
    _epilogue_body,
    grid=(_B,),
    in_specs=[
        pl.BlockSpec((2, 1, 8, 128), lambda b: (0, b, 0, 0)),
        pl.BlockSpec((2, 1, _C, _S, _CB), lambda b: (0, b, 0, 0, 0)),
        pl.BlockSpec((2, 1, _C * _GP, _S, 2 * _TB), lambda b: (0, b, 0, 0, 0)),
    ],
    out_specs=[
        pl.BlockSpec((1, 8, 128), lambda b: (b, 0, 0)),
        pl.BlockSpec((1, _S, _C * _CB), lambda b: (b, 0, 0)),
        pl.BlockSpec((1, _S, _C * _G * _TB), lambda b: (b, 0, 0)),
    ],
    out_shape=[
        jax.ShapeDtypeStruct((_B, 8, 128), jnp.float32),
        jax.ShapeDtypeStruct((_B, _S, _C * _CB), jnp.float32),
        jax.ShapeDtypeStruct((_B, _S, _C * _G * _TB), jnp.float32),
    ],
)


def kernel(imgs, grads, reg_lab):
    B, C = imgs.shape[0], imgs.shape[1]
    G = grads.shape[2]
    imgs2 = imgs.reshape(B * C, _HW)
    grads2 = grads.reshape(B * C * G, _HW)
    rl2 = reg_lab.astype(jnp.int32).reshape(B, _HW)

    rs2, ch2, th2 = _sc_hist(imgs2, grads2, rl2)

    rs, ch, th = _epilogue(rs2.reshape(2, B, 8, 128),
                           ch2.reshape(2, B, C, _S, _CB),
                           th2.reshape(2, B, C * _GP, _S, 2 * _TB))
    return jnp.concatenate(
        [rs.reshape(B, _S), ch.reshape(B, -1), th.reshape(B, -1)], axis=-1)
